# Initial kernel scaffold; baseline (speedup 1.0000x reference)
import jax
import jax.numpy as jnp
from jax.experimental import pallas as pl


def kernel(input_src, h0, c0, embed, W_ih, W_hh, b_ih, b_hh):
    raise NotImplementedError("write your pallas kernel here")


# speedup vs baseline: 1.8116x; 1.8116x over previous
import functools

import jax
import jax.numpy as jnp
from jax import lax
from jax.experimental import pallas as pl
from jax.experimental.pallas import tpu as pltpu
from jax.experimental.pallas import tpu_sc as plsc


def _make_sc_gather(V, D, N, NC, NS, chunk):
    NW = NC * NS
    n_per_w = N // NW
    n_ch = n_per_w // chunk
    mesh = plsc.VectorSubcoreMesh(core_axis_name="c", subcore_axis_name="s")

    @functools.partial(
        pl.kernel,
        mesh=mesh,
        compiler_params=pltpu.CompilerParams(use_tc_tiling_on_sc=False),
        out_type=jax.ShapeDtypeStruct((N, D), jnp.float32),
        scratch_types=[
            pltpu.VMEM((n_ch, chunk), jnp.int32),
            pltpu.VMEM((chunk, D), jnp.float32),
            pltpu.SemaphoreType.DMA,
        ],
    )
    def gather_k(table_hbm, idx_hbm, out_hbm, idx_v, rows_v, sem):
        wid = lax.axis_index("s") * NC + lax.axis_index("c")
        pltpu.sync_copy(idx_hbm.at[wid], idx_v)
        base = wid * n_per_w

        def body(j, _):
            pltpu.async_copy(table_hbm.at[idx_v.at[j]], rows_v, sem).wait()
            pltpu.sync_copy(rows_v, out_hbm.at[pl.ds(base + j * chunk, chunk)])
            return 0

        lax.fori_loop(0, n_ch, body, 0)

    return gather_k


def _lstm_body(Hd, T, emb_ref, h0_ref, c0_ref, wih_ref, whh_ref, b_ref,
               out_ref, hT_ref, cT_ref, h_s, c_s):
    t = pl.program_id(0)

    @pl.when(t == 0)
    def _():
        h_s[...] = h0_ref[0]
        c_s[...] = c0_ref[0]

    x = emb_ref[0]
    h = h_s[...]
    gates = jnp.dot(x, wih_ref[...], preferred_element_type=jnp.float32)
    gates = gates + jnp.dot(h, whh_ref[...], preferred_element_type=jnp.float32)
    gates = gates + b_ref[...]
    i = jax.nn.sigmoid(gates[:, 0 * Hd:1 * Hd])
    f = jax.nn.sigmoid(gates[:, 1 * Hd:2 * Hd])
    g = jnp.tanh(gates[:, 2 * Hd:3 * Hd])
    o = jax.nn.sigmoid(gates[:, 3 * Hd:4 * Hd])
    c = f * c_s[...] + i * g
    h_new = o * jnp.tanh(c)
    h_s[...] = h_new
    c_s[...] = c
    out_ref[0] = h_new

    @pl.when(t == T - 1)
    def _():
        hT_ref[0] = h_new
        cT_ref[0] = c


def _lstm_call(emb, h0, c0, wihT, whhT, bias, interpret=False):
    T, B, Hd = emb.shape
    grid = (T,)
    out_shapes = (
        jax.ShapeDtypeStruct((T, B, Hd), jnp.float32),
        jax.ShapeDtypeStruct((1, B, Hd), jnp.float32),
        jax.ShapeDtypeStruct((1, B, Hd), jnp.float32),
    )
    return pl.pallas_call(
        functools.partial(_lstm_body, Hd, T),
        grid=grid,
        in_specs=[
            pl.BlockSpec((1, B, Hd), lambda t: (t, 0, 0)),
            pl.BlockSpec((1, B, Hd), lambda t: (0, 0, 0)),
            pl.BlockSpec((1, B, Hd), lambda t: (0, 0, 0)),
            pl.BlockSpec((Hd, 4 * Hd), lambda t: (0, 0)),
            pl.BlockSpec((Hd, 4 * Hd), lambda t: (0, 0)),
            pl.BlockSpec((1, 4 * Hd), lambda t: (0, 0)),
        ],
        out_specs=(
            pl.BlockSpec((1, B, Hd), lambda t: (t, 0, 0)),
            pl.BlockSpec((1, B, Hd), lambda t: (0, 0, 0)),
            pl.BlockSpec((1, B, Hd), lambda t: (0, 0, 0)),
        ),
        out_shape=out_shapes,
        scratch_shapes=[
            pltpu.VMEM((B, Hd), jnp.float32),
            pltpu.VMEM((B, Hd), jnp.float32),
        ],
        interpret=interpret,
    )(emb, h0, c0, wihT, whhT, bias)


def kernel(input_src, h0, c0, embed, W_ih, W_hh, b_ih, b_hh):
    T, B = input_src.shape
    V, Hd = embed.shape
    N = T * B
    chunk = 80

    info = plsc.get_sparse_core_info()
    NC, NS = info.num_cores, info.num_subcores

    NW = NC * NS
    idx = input_src.reshape(NW, N // (NW * chunk), chunk).astype(jnp.int32)
    gather = _make_sc_gather(V, Hd, N, NC, NS, chunk)
    emb_flat = gather(embed, idx)
    emb = emb_flat.reshape(T, B, Hd)

    wihT = W_ih.T
    whhT = W_hh.T
    bias = (b_ih + b_hh).reshape(1, 4 * Hd)
    out, hT, cT = _lstm_call(emb, h0, c0, wihT, whhT, bias)
    return out, (hT, cT)


# speedup vs baseline: 2.3316x; 1.2870x over previous
import functools

import jax
import jax.numpy as jnp
from jax import lax
from jax.experimental import pallas as pl
from jax.experimental.pallas import tpu as pltpu
from jax.experimental.pallas import tpu_sc as plsc


def _make_sc_gather(V, D, N, NC, NS, chunk):
    NW = NC * NS
    n_per_w = N // NW
    n_ch = n_per_w // chunk
    mesh = plsc.VectorSubcoreMesh(core_axis_name="c", subcore_axis_name="s")

    @functools.partial(
        pl.kernel,
        mesh=mesh,
        compiler_params=pltpu.CompilerParams(use_tc_tiling_on_sc=False),
        out_type=jax.ShapeDtypeStruct((N, D), jnp.float32),
        scratch_types=[
            pltpu.VMEM((n_ch, chunk), jnp.int32),
            pltpu.VMEM((chunk, D), jnp.float32),
            pltpu.SemaphoreType.DMA,
        ],
    )
    def gather_k(table_hbm, idx_hbm, out_hbm, idx_v, rows_v, sem):
        wid = lax.axis_index("s") * NC + lax.axis_index("c")
        pltpu.sync_copy(idx_hbm.at[wid], idx_v)
        base = wid * n_per_w

        def body(j, _):
            pltpu.async_copy(table_hbm.at[idx_v.at[j]], rows_v, sem).wait()
            pltpu.sync_copy(rows_v, out_hbm.at[pl.ds(base + j * chunk, chunk)])
            return 0

        lax.fori_loop(0, n_ch, body, 0)

    return gather_k


def _lstm_body(Hd, T, x_ref, h0_ref, c0_ref, wih_ref, whh_ref, b_ref,
               out_ref, hT_ref, cT_ref, h_s, c_s):
    t = pl.program_id(0)

    @pl.when(t == 0)
    def _():
        h_s[...] = h0_ref[...]
        c_s[...] = c0_ref[...]

    x2 = x_ref[0]
    xe = x2[:, 0:Hd]
    xo = x2[:, Hd:2 * Hd]
    wih = wih_ref[...]
    dn = (((1,), (1,)), ((), ()))
    ge = lax.dot_general(wih, xe, dn, preferred_element_type=jnp.float32)
    go = lax.dot_general(wih, xo, dn, preferred_element_type=jnp.float32)
    gx = jnp.concatenate([ge, go], axis=1)
    h = h_s[...]
    gates = gx + jnp.dot(whh_ref[...], h, preferred_element_type=jnp.float32)
    gates = gates + b_ref[...]
    i = jax.nn.sigmoid(gates[0 * Hd:1 * Hd, :])
    f = jax.nn.sigmoid(gates[1 * Hd:2 * Hd, :])
    g = jnp.tanh(gates[2 * Hd:3 * Hd, :])
    o = jax.nn.sigmoid(gates[3 * Hd:4 * Hd, :])
    c = f * c_s[...] + i * g
    h_new = o * jnp.tanh(c)
    h_s[...] = h_new
    c_s[...] = c
    out_ref[0] = h_new

    @pl.when(t == T - 1)
    def _():
        hT_ref[...] = h_new
        cT_ref[...] = c


def _lstm_call(x2, h0t, c0t, wih, whh, bias, interpret=False):
    T, Bh, Hd2 = x2.shape
    Hd = Hd2 // 2
    B = Bh * 2
    out_shapes = (
        jax.ShapeDtypeStruct((T, Hd, B), jnp.float32),
        jax.ShapeDtypeStruct((Hd, B), jnp.float32),
        jax.ShapeDtypeStruct((Hd, B), jnp.float32),
    )
    return pl.pallas_call(
        functools.partial(_lstm_body, Hd, T),
        grid=(T,),
        in_specs=[
            pl.BlockSpec((1, Bh, Hd2), lambda t: (t, 0, 0)),
            pl.BlockSpec((Hd, B), lambda t: (0, 0)),
            pl.BlockSpec((Hd, B), lambda t: (0, 0)),
            pl.BlockSpec((4 * Hd, Hd), lambda t: (0, 0)),
            pl.BlockSpec((4 * Hd, Hd), lambda t: (0, 0)),
            pl.BlockSpec((4 * Hd, 1), lambda t: (0, 0)),
        ],
        out_specs=(
            pl.BlockSpec((1, Hd, B), lambda t: (t, 0, 0)),
            pl.BlockSpec((Hd, B), lambda t: (0, 0)),
            pl.BlockSpec((Hd, B), lambda t: (0, 0)),
        ),
        out_shape=out_shapes,
        scratch_shapes=[
            pltpu.VMEM((Hd, B), jnp.float32),
            pltpu.VMEM((Hd, B), jnp.float32),
        ],
        interpret=interpret,
    )(x2, h0t, c0t, wih, whh, bias)


def kernel(input_src, h0, c0, embed, W_ih, W_hh, b_ih, b_hh):
    T, B = input_src.shape
    V, Hd = embed.shape
    N = T * B
    Bh = B // 2
    chunk = 80

    info = plsc.get_sparse_core_info()
    NC, NS = info.num_cores, info.num_subcores
    NW = NC * NS

    idx2 = input_src.reshape(T, 2, Bh).transpose(0, 2, 1)
    idx2 = idx2.reshape(NW, N // (NW * chunk), chunk).astype(jnp.int32)

    gather = _make_sc_gather(V, Hd, N, NC, NS, chunk)
    emb_flat = gather(embed, idx2)
    x2 = emb_flat.reshape(T, Bh, 2 * Hd)

    h0t = h0[0].T
    c0t = c0[0].T
    bias = (b_ih + b_hh).reshape(4 * Hd, 1)
    out_t, hTt, cTt = _lstm_call(x2, h0t, c0t, W_ih, W_hh, bias)

    out = out_t.transpose(0, 2, 1)
    hT = hTt.T[None]
    cT = cTt.T[None]
    return out, (hT, cT)
